# baseline (speedup 1.0000x reference)
import jax
import jax.numpy as jnp
from jax import lax
from jax.experimental import pallas as pl
from jax.experimental.pallas import tpu as pltpu

_BR = 8


def _body(addval_ref, diff_ref, w_ref, o_ref):
    b, n = diff_ref.shape
    a0 = jnp.abs(diff_ref[...])
    idx = lax.broadcasted_iota(jnp.int32, (b, n), 1)
    a = a0
    for _ in range(3):
        m = jnp.max(a, axis=1, keepdims=True)
        gi = jnp.min(jnp.where(a == m, idx, n), axis=1, keepdims=True)
        a = jnp.where(idx == gi, -1.0, a)
    o_ref[...] = w_ref[...] + jnp.where(a != a0, addval_ref[0], 0.0)


def kernel(difference, weight, epoch):
    b, n = difference.shape
    cond = (200 < epoch) & (epoch < 1000) & (epoch % 20 == 0)
    addval = jnp.where(cond, jnp.float32(1.0), jnp.float32(0.0)).reshape(1)
    br = _BR if b % _BR == 0 else b
    return pl.pallas_call(
        _body,
        grid=(b // br,),
        out_shape=jax.ShapeDtypeStruct((b, n), jnp.float32),
        in_specs=[
            pl.BlockSpec(memory_space=pltpu.SMEM),
            pl.BlockSpec((br, n), lambda i: (i, 0)),
            pl.BlockSpec((br, n), lambda i: (i, 0)),
        ],
        out_specs=pl.BlockSpec((br, n), lambda i: (i, 0)),
    )(addval, difference, weight)


# speedup vs baseline: 1.6505x; 1.6505x over previous
import jax
import jax.numpy as jnp
from jax import lax
from jax.experimental import pallas as pl
from jax.experimental.pallas import tpu as pltpu


def _body(addval_ref, diff_ref, w_ref, o_ref):
    b, n = diff_ref.shape
    a = jnp.abs(diff_ref[...])
    m1 = jnp.max(a, axis=1, keepdims=True)
    m2 = jnp.max(jnp.where(a == m1, -1.0, a), axis=1, keepdims=True)
    m3 = jnp.max(jnp.where(a >= m2, -1.0, a), axis=1, keepdims=True)
    ge3 = a >= m3
    cnt = jnp.sum(ge3.astype(jnp.int32), axis=1, keepdims=True)
    allok = jnp.all(cnt == 3)
    addv = addval_ref[0]

    @pl.when(allok)
    def _fast():
        o_ref[...] = w_ref[...] + jnp.where(ge3, addv, 0.0)

    @pl.when(jnp.logical_not(allok))
    def _exact():
        idx = lax.broadcasted_iota(jnp.int32, (b, n), 1)
        av = a
        mask = jnp.zeros((b, n), jnp.bool_)
        for _ in range(3):
            m = jnp.max(av, axis=1, keepdims=True)
            gi = jnp.min(jnp.where(av == m, idx, n), axis=1, keepdims=True)
            sel = idx == gi
            mask = mask | sel
            av = jnp.where(sel, -1.0, av)
        o_ref[...] = w_ref[...] + jnp.where(mask, addv, 0.0)


def kernel(difference, weight, epoch):
    b, n = difference.shape
    cond = (200 < epoch) & (epoch < 1000) & (epoch % 20 == 0)
    addval = jnp.where(cond, jnp.float32(1.0), jnp.float32(0.0)).reshape(1)
    return pl.pallas_call(
        _body,
        out_shape=jax.ShapeDtypeStruct((b, n), jnp.float32),
        in_specs=[
            pl.BlockSpec(memory_space=pltpu.SMEM),
            pl.BlockSpec((b, n), lambda: (0, 0)),
            pl.BlockSpec((b, n), lambda: (0, 0)),
        ],
        out_specs=pl.BlockSpec((b, n), lambda: (0, 0)),
    )(addval, difference, weight)
